# baseline (speedup 1.0000x reference)
import functools

import jax
import jax.numpy as jnp
from jax import lax
from jax.experimental import pallas as pl
from jax.experimental.pallas import tpu as pltpu
from jax.experimental.pallas import tpu_sc as plsc

D = 128
NG = 100000
CHUNK = 4096
NPASS = 13
NPAD = 2 * NPASS * CHUNK
ACC_ROWS = CHUNK + 8
E = 200000
EPT = 12544
EPAD = 16 * EPT
NB = 128
RING = 512
RMASK = RING - 1
SENT = 1 << 28
ROWS_TILE = CHUNK // 16


def _proj_body(x_ref, w_ref, b_ref, o_ref):
    o_ref[...] = (
        jnp.dot(x_ref[...], w_ref[...], preferred_element_type=jnp.float32)
        + b_ref[...]
    )


def _proj(x, W, b, bm=1000):
    n, fin = x.shape
    return pl.pallas_call(
        _proj_body,
        grid=(n // bm,),
        in_specs=[
            pl.BlockSpec((bm, fin), lambda i: (i, 0)),
            pl.BlockSpec((fin, D), lambda i: (0, 0)),
            pl.BlockSpec((1, D), lambda i: (0, 0)),
        ],
        out_specs=pl.BlockSpec((bm, D), lambda i: (i, 0)),
        out_shape=jax.ShapeDtypeStruct((n, D), jnp.float32),
    )(x, W, b.reshape(1, D))


@functools.partial(
    pl.kernel,
    out_type=[
        jax.ShapeDtypeStruct((NPAD, D), jnp.float32),
        jax.ShapeDtypeStruct((NPAD, D), jnp.float32),
        jax.ShapeDtypeStruct((NPAD, D), jnp.float32),
        jax.ShapeDtypeStruct((NPAD, D), jnp.float32),
        jax.ShapeDtypeStruct((NPAD, D), jnp.float32),
        jax.ShapeDtypeStruct((NPAD, D), jnp.float32),
    ],
    mesh=plsc.VectorSubcoreMesh(core_axis_name="c", subcore_axis_name="s"),
    compiler_params=pltpu.CompilerParams(needs_layout_passes=False),
    scratch_types=[
        pltpu.VMEM((EPT,), jnp.int32),
        pltpu.VMEM((EPT,), jnp.int32),
        pltpu.VMEM((RING,), jnp.int32),
        pltpu.VMEM((RING,), jnp.int32),
        pltpu.VMEM((NB, D), jnp.float32),
        pltpu.VMEM((NB, D), jnp.float32),
        pltpu.VMEM((NB,), jnp.int32),
        pltpu.VMEM((NB,), jnp.int32),
        pltpu.VMEM((NB,), jnp.int32),
        pltpu.VMEM_SHARED((2 * ACC_ROWS, D), jnp.float32),
        pltpu.SemaphoreType.DMA,
    ],
)
def _sc_segsum(
    src_tg, dst_tg, tab_tg, src_ag, dst_ag, tab_ag, src_pg, dst_pg, tab_pg,
    sum_tg, cnt_tg, sum_ag, cnt_ag, sum_pg, cnt_pg,
    src_raw, dst_raw, src_ring, dst_ring, rows_v, ones_v, srcb_v, dstb_v,
    dstb2_v, acc_sh, sem,
):
    cid = lax.axis_index("c")
    sid = lax.axis_index("s")
    ebase = sid * EPT
    rbase = sid * ROWS_TILE

    def _init(i, _):
        for j in range(D // 16):
            ones_v[i, pl.ds(j * 16, 16)] = jnp.ones((16,), jnp.float32)
        return 0

    lax.fori_loop(0, NB, _init, 0)

    rels = (
        (src_tg, dst_tg, tab_tg, sum_tg, cnt_tg),
        (src_ag, dst_ag, tab_ag, sum_ag, cnt_ag),
        (src_pg, dst_pg, tab_pg, sum_pg, cnt_pg),
    )
    for src_h, dst_h, tab_h, sum_h, cnt_h in rels:
        pltpu.sync_copy(src_h.at[pl.ds(ebase, EPT)], src_raw)
        pltpu.sync_copy(dst_h.at[pl.ds(ebase, EPT)], dst_raw)
        for p in range(NPASS):
            lo = (cid * NPASS + p) * CHUNK

            def _zrow(i, _):
                for j in range(D // 16):
                    rows_v[i, pl.ds(j * 16, 16)] = jnp.zeros((16,), jnp.float32)
                return 0

            lax.fori_loop(0, NB, _zrow, 0)
            for j in range(ROWS_TILE // NB):
                pltpu.sync_copy(rows_v, acc_sh.at[pl.ds(rbase + j * NB, NB)])
                pltpu.sync_copy(
                    rows_v, acc_sh.at[pl.ds(ACC_ROWS + rbase + j * NB, NB)]
                )
            plsc.subcore_barrier()

            lov = lax.broadcast(lo, (16,))

            def _fire(nf):
                base = nf & RMASK
                for j in range(NB // 16):
                    srcb_v[pl.ds(j * 16, 16)] = src_ring[pl.ds(base + j * 16, 16)]
                    dloc = dst_ring[pl.ds(base + j * 16, 16)]
                    dstb_v[pl.ds(j * 16, 16)] = dloc
                    dstb2_v[pl.ds(j * 16, 16)] = dloc + ACC_ROWS
                pltpu.async_copy(tab_h.at[srcb_v], rows_v, sem).wait()
                pltpu.sync_copy(rows_v, acc_sh.at[dstb_v], add=True)
                pltpu.sync_copy(ones_v, acc_sh.at[dstb2_v], add=True)

            def _scan(i, carry):
                k, nf = carry
                sv = src_raw[pl.ds(i * 16, 16)]
                dv = dst_raw[pl.ds(i * 16, 16)]
                dl = dv - lov
                m = (dl >= 0) & (dl < CHUNK)
                cum = plsc.cumsum(m.astype(jnp.int32))
                pos = lax.broadcast(k - 1, (16,)) + cum
                plsc.store_scatter(src_ring, [pos & RMASK], sv, mask=m)
                plsc.store_scatter(dst_ring, [pos & RMASK], dl, mask=m)
                k2 = k + cum[15]
                fire_p = (k2 - nf) >= NB

                @pl.when(fire_p)
                def _():
                    _fire(nf)

                nf2 = jnp.where(fire_p, nf + NB, nf)
                return (k2, nf2)

            k, nf = lax.fori_loop(0, EPT // 16, _scan, (0, 0))

            iota = lax.iota(jnp.int32, 16)
            for j in range(NB // 16):
                padpos = (lax.broadcast(k + j * 16, (16,)) + iota) & RMASK
                plsc.store_scatter(src_ring, [padpos], jnp.zeros((16,), jnp.int32))
                plsc.store_scatter(
                    dst_ring, [padpos], jnp.full((16,), CHUNK, jnp.int32)
                )
            _fire(nf)
            plsc.subcore_barrier()
            pltpu.sync_copy(
                acc_sh.at[pl.ds(rbase, ROWS_TILE)],
                sum_h.at[pl.ds(lo + rbase, ROWS_TILE)],
            )
            pltpu.sync_copy(
                acc_sh.at[pl.ds(ACC_ROWS + rbase, ROWS_TILE)],
                cnt_h.at[pl.ds(lo + rbase, ROWS_TILE)],
            )
            plsc.subcore_barrier()


def _dot(a, b):
    return jnp.dot(a, b, preferred_element_type=jnp.float32)


def _final_body(
    xg_ref, st_ref, ct_ref, sa_ref, ca_ref, sp_ref, cp_ref,
    wlt_ref, wla_ref, wlp_ref, wrt_ref, wra_ref, wrp_ref, bs_ref,
    wd1_ref, bd1_ref, wd2_ref, bd2_ref,
    wv1_ref, bv1_ref, wv2_ref, bv2_ref,
    p_ref, v_ref,
):
    def mean(s_ref, c_ref):
        c = c_ref[...][:, :1]
        return s_ref[...] / jnp.maximum(c, 1.0)

    xg = xg_ref[...]
    out = _dot(mean(st_ref, ct_ref), wlt_ref[...]) + _dot(xg, wrt_ref[...]) + bs_ref[...]
    out = out + _dot(mean(sa_ref, ca_ref), wla_ref[...]) + _dot(xg, wra_ref[...])
    out = out + _dot(mean(sp_ref, cp_ref), wlp_ref[...]) + _dot(xg, wrp_ref[...])
    hd = jax.nn.relu(_dot(out, wd1_ref[...]) + bd1_ref[...])
    p_ref[...] = jax.nn.sigmoid(_dot(hd, wd2_ref[...]) + bd2_ref[...])
    hv = jax.nn.relu(_dot(out, wv1_ref[...]) + bv1_ref[...])
    v_ref[...] = _dot(hv, wv2_ref[...]) + bv2_ref[...]


def _final(xg, st, ct, sa, ca, sp, cp, wlt, wla, wlp, wrt, wra, wrp, bs,
           wd1, bd1, wd2, bd2, wv1, bv1, wv2, bv2, bm=1000):
    full = lambda shape: pl.BlockSpec(shape, lambda i: (0, 0))
    row = lambda shape: pl.BlockSpec(shape, lambda i: (i, 0))
    return pl.pallas_call(
        _final_body,
        grid=(NG // bm,),
        in_specs=[
            row((bm, D)), row((bm, D)), row((bm, D)), row((bm, D)),
            row((bm, D)), row((bm, D)), row((bm, D)),
            full((D, D)), full((D, D)), full((D, D)), full((D, D)),
            full((D, D)), full((D, D)), full((1, D)),
            full((D, 64)), full((1, 64)), full((64, 1)), full((1, 1)),
            full((D, 64)), full((1, 64)), full((64, 1)), full((1, 1)),
        ],
        out_specs=[row((bm, 1)), row((bm, 1))],
        out_shape=[
            jax.ShapeDtypeStruct((NG, 1), jnp.float32),
            jax.ShapeDtypeStruct((NG, 1), jnp.float32),
        ],
    )(xg, st, ct, sa, ca, sp, cp, wlt, wla, wlp, wrt, wra, wrp, bs,
      wd1, bd1, wd2, bd2, wv1, bv1, wv2, bv2)


def _prep_edges(ei):
    src = ei[0].astype(jnp.int32)
    dst = ei[1].astype(jnp.int32)
    pad = EPAD - src.shape[0]
    src = jnp.pad(src, (0, pad))
    dst = jnp.pad(dst, (0, pad), constant_values=SENT)
    return src, dst


def kernel(x_gene, x_atac, x_tad, x_protein, ei_tad_gene, ei_atac_gene,
           ei_prot_gene, x_map_gene, Win_gene, bin_gene, Win_atac, bin_atac,
           Wl_tg, Wr_tg, b_tg, Wl_ag, Wr_ag, b_ag, Wl_pg, Wr_pg, b_pg,
           Wd1, bd1, Wd2, bd2, Wv1, bv1, Wv2, bv2):
    xg = _proj(x_gene, Win_gene, bin_gene)
    xa = _proj(x_atac, Win_atac, bin_atac)
    stg, dtg = _prep_edges(ei_tad_gene)
    sag, dag = _prep_edges(ei_atac_gene)
    spg, dpg = _prep_edges(ei_prot_gene)
    sum_tg, cnt_tg, sum_ag, cnt_ag, sum_pg, cnt_pg = _sc_segsum(
        stg, dtg, x_tad, sag, dag, xa, spg, dpg, x_protein
    )
    bs = (b_tg + b_ag + b_pg).reshape(1, D)
    p, v = _final(
        xg, sum_tg[:NG], cnt_tg[:NG], sum_ag[:NG], cnt_ag[:NG],
        sum_pg[:NG], cnt_pg[:NG],
        Wl_tg, Wl_ag, Wl_pg, Wr_tg, Wr_ag, Wr_pg, bs,
        Wd1, bd1.reshape(1, 64), Wd2, bd2.reshape(1, 1),
        Wv1, bv1.reshape(1, 64), Wv2, bv2.reshape(1, 1),
    )
    dropouts = jax.random.bernoulli(jax.random.key(1), p).astype(jnp.float32)
    return (p, dropouts, v, x_map_gene)


# speedup vs baseline: 1.0157x; 1.0157x over previous
import functools

import jax
import jax.numpy as jnp
from jax import lax
from jax.experimental import pallas as pl
from jax.experimental.pallas import tpu as pltpu
from jax.experimental.pallas import tpu_sc as plsc

D = 128
NG = 100000
CHUNK = 4096
NPASS = 13
NPAD = 2 * NPASS * CHUNK
ACC_ROWS = CHUNK + 8
E = 200000
EPT = 12544
EPAD = 16 * EPT
NB = 128
RING = 512
RMASK = RING - 1
SENT = 1 << 28
ROWS_TILE = CHUNK // 16


def _proj_body(x_ref, w_ref, b_ref, o_ref):
    o_ref[...] = (
        jnp.dot(x_ref[...], w_ref[...], preferred_element_type=jnp.float32)
        + b_ref[...]
    )


def _proj(x, W, b, bm=1000):
    n, fin = x.shape
    return pl.pallas_call(
        _proj_body,
        grid=(n // bm,),
        in_specs=[
            pl.BlockSpec((bm, fin), lambda i: (i, 0)),
            pl.BlockSpec((fin, D), lambda i: (0, 0)),
            pl.BlockSpec((1, D), lambda i: (0, 0)),
        ],
        out_specs=pl.BlockSpec((bm, D), lambda i: (i, 0)),
        out_shape=jax.ShapeDtypeStruct((n, D), jnp.float32),
    )(x, W, b.reshape(1, D))


@functools.partial(
    pl.kernel,
    out_type=[
        jax.ShapeDtypeStruct((NPAD, D), jnp.float32),
        jax.ShapeDtypeStruct((NPAD, D), jnp.float32),
        jax.ShapeDtypeStruct((NPAD, D), jnp.float32),
        jax.ShapeDtypeStruct((NPAD, D), jnp.float32),
        jax.ShapeDtypeStruct((NPAD, D), jnp.float32),
        jax.ShapeDtypeStruct((NPAD, D), jnp.float32),
    ],
    mesh=plsc.VectorSubcoreMesh(core_axis_name="c", subcore_axis_name="s"),
    compiler_params=pltpu.CompilerParams(needs_layout_passes=False),
    scratch_types=[
        pltpu.VMEM((EPT,), jnp.int32),
        pltpu.VMEM((EPT,), jnp.int32),
        pltpu.VMEM((RING,), jnp.int32),
        pltpu.VMEM((RING,), jnp.int32),
        pltpu.VMEM((NB, D), jnp.float32),
        pltpu.VMEM((NB, D), jnp.float32),
        pltpu.VMEM((NB,), jnp.int32),
        pltpu.VMEM((NB,), jnp.int32),
        pltpu.VMEM((NB,), jnp.int32),
        pltpu.VMEM_SHARED((2 * ACC_ROWS, D), jnp.float32),
        pltpu.SemaphoreType.DMA,
        pltpu.SemaphoreType.DMA,
    ],
)
def _sc_segsum(
    src_tg, dst_tg, tab_tg, src_ag, dst_ag, tab_ag, src_pg, dst_pg, tab_pg,
    sum_tg, cnt_tg, sum_ag, cnt_ag, sum_pg, cnt_pg,
    src_raw, dst_raw, src_ring, dst_ring, rows_v, ones_v, srcb_v, dstb_v,
    dstb2_v, acc_sh, sem, sem2,
):
    cid = lax.axis_index("c")
    sid = lax.axis_index("s")
    ebase = sid * EPT
    rbase = sid * ROWS_TILE

    def _init(i, _):
        for j in range(D // 16):
            ones_v[i, pl.ds(j * 16, 16)] = jnp.ones((16,), jnp.float32)
        return 0

    lax.fori_loop(0, NB, _init, 0)

    rels = (
        (src_tg, dst_tg, tab_tg, sum_tg, cnt_tg),
        (src_ag, dst_ag, tab_ag, sum_ag, cnt_ag),
        (src_pg, dst_pg, tab_pg, sum_pg, cnt_pg),
    )
    for src_h, dst_h, tab_h, sum_h, cnt_h in rels:
        st1 = pltpu.async_copy(src_h.at[pl.ds(ebase, EPT)], src_raw, sem2)
        pltpu.sync_copy(dst_h.at[pl.ds(ebase, EPT)], dst_raw)
        st1.wait()
        for p in range(NPASS):
            lo = (cid * NPASS + p) * CHUNK

            def _zrow(i, _):
                for j in range(D // 16):
                    rows_v[i, pl.ds(j * 16, 16)] = jnp.zeros((16,), jnp.float32)
                return 0

            lax.fori_loop(0, NB, _zrow, 0)
            for j in range(ROWS_TILE // NB):
                pltpu.sync_copy(rows_v, acc_sh.at[pl.ds(rbase + j * NB, NB)])
                pltpu.sync_copy(
                    rows_v, acc_sh.at[pl.ds(ACC_ROWS + rbase + j * NB, NB)]
                )
            plsc.subcore_barrier()

            lov = lax.broadcast(lo, (16,))

            def _fire(nf):
                base = nf & RMASK
                for j in range(NB // 16):
                    srcb_v[pl.ds(j * 16, 16)] = src_ring[pl.ds(base + j * 16, 16)]
                    dloc = dst_ring[pl.ds(base + j * 16, 16)]
                    dstb_v[pl.ds(j * 16, 16)] = dloc
                    dstb2_v[pl.ds(j * 16, 16)] = dloc + ACC_ROWS
                cntd = pltpu.async_copy(
                    ones_v, acc_sh.at[dstb2_v], sem2, add=True
                )
                pltpu.async_copy(tab_h.at[srcb_v], rows_v, sem).wait()
                pltpu.sync_copy(rows_v, acc_sh.at[dstb_v], add=True)
                cntd.wait()

            def _scan(i, carry):
                k, nf = carry
                sv = src_raw[pl.ds(i * 16, 16)]
                dv = dst_raw[pl.ds(i * 16, 16)]
                dl = dv - lov
                m = (dl >= 0) & (dl < CHUNK)
                cum = plsc.cumsum(m.astype(jnp.int32))
                pos = lax.broadcast(k - 1, (16,)) + cum
                plsc.store_scatter(src_ring, [pos & RMASK], sv, mask=m)
                plsc.store_scatter(dst_ring, [pos & RMASK], dl, mask=m)
                k2 = k + cum[15]
                fire_p = (k2 - nf) >= NB

                @pl.when(fire_p)
                def _():
                    _fire(nf)

                nf2 = jnp.where(fire_p, nf + NB, nf)
                return (k2, nf2)

            k, nf = lax.fori_loop(0, EPT // 16, _scan, (0, 0))

            iota = lax.iota(jnp.int32, 16)
            for j in range(NB // 16):
                padpos = (lax.broadcast(k + j * 16, (16,)) + iota) & RMASK
                plsc.store_scatter(src_ring, [padpos], jnp.zeros((16,), jnp.int32))
                plsc.store_scatter(
                    dst_ring, [padpos], jnp.full((16,), CHUNK, jnp.int32)
                )
            _fire(nf)
            plsc.subcore_barrier()
            pltpu.sync_copy(
                acc_sh.at[pl.ds(rbase, ROWS_TILE)],
                sum_h.at[pl.ds(lo + rbase, ROWS_TILE)],
            )
            pltpu.sync_copy(
                acc_sh.at[pl.ds(ACC_ROWS + rbase, ROWS_TILE)],
                cnt_h.at[pl.ds(lo + rbase, ROWS_TILE)],
            )
            plsc.subcore_barrier()


def _dot(a, b):
    return jnp.dot(a, b, preferred_element_type=jnp.float32)


def _final_body(
    xg_ref, st_ref, ct_ref, sa_ref, ca_ref, sp_ref, cp_ref,
    wlt_ref, wla_ref, wlp_ref, wrt_ref, wra_ref, wrp_ref, bs_ref,
    wd1_ref, bd1_ref, wd2_ref, bd2_ref,
    wv1_ref, bv1_ref, wv2_ref, bv2_ref,
    p_ref, v_ref,
):
    def mean(s_ref, c_ref):
        c = c_ref[...][:, :1]
        return s_ref[...] / jnp.maximum(c, 1.0)

    xg = xg_ref[...]
    out = _dot(mean(st_ref, ct_ref), wlt_ref[...]) + _dot(xg, wrt_ref[...]) + bs_ref[...]
    out = out + _dot(mean(sa_ref, ca_ref), wla_ref[...]) + _dot(xg, wra_ref[...])
    out = out + _dot(mean(sp_ref, cp_ref), wlp_ref[...]) + _dot(xg, wrp_ref[...])
    hd = jax.nn.relu(_dot(out, wd1_ref[...]) + bd1_ref[...])
    p_ref[...] = jax.nn.sigmoid(_dot(hd, wd2_ref[...]) + bd2_ref[...])
    hv = jax.nn.relu(_dot(out, wv1_ref[...]) + bv1_ref[...])
    v_ref[...] = _dot(hv, wv2_ref[...]) + bv2_ref[...]


def _final(xg, st, ct, sa, ca, sp, cp, wlt, wla, wlp, wrt, wra, wrp, bs,
           wd1, bd1, wd2, bd2, wv1, bv1, wv2, bv2, bm=1000):
    full = lambda shape: pl.BlockSpec(shape, lambda i: (0, 0))
    row = lambda shape: pl.BlockSpec(shape, lambda i: (i, 0))
    return pl.pallas_call(
        _final_body,
        grid=(NG // bm,),
        in_specs=[
            row((bm, D)), row((bm, D)), row((bm, D)), row((bm, D)),
            row((bm, D)), row((bm, D)), row((bm, D)),
            full((D, D)), full((D, D)), full((D, D)), full((D, D)),
            full((D, D)), full((D, D)), full((1, D)),
            full((D, 64)), full((1, 64)), full((64, 1)), full((1, 1)),
            full((D, 64)), full((1, 64)), full((64, 1)), full((1, 1)),
        ],
        out_specs=[row((bm, 1)), row((bm, 1))],
        out_shape=[
            jax.ShapeDtypeStruct((NG, 1), jnp.float32),
            jax.ShapeDtypeStruct((NG, 1), jnp.float32),
        ],
    )(xg, st, ct, sa, ca, sp, cp, wlt, wla, wlp, wrt, wra, wrp, bs,
      wd1, bd1, wd2, bd2, wv1, bv1, wv2, bv2)


def _prep_edges(ei):
    src = ei[0].astype(jnp.int32)
    dst = ei[1].astype(jnp.int32)
    pad = EPAD - src.shape[0]
    src = jnp.pad(src, (0, pad))
    dst = jnp.pad(dst, (0, pad), constant_values=SENT)
    return src, dst


def kernel(x_gene, x_atac, x_tad, x_protein, ei_tad_gene, ei_atac_gene,
           ei_prot_gene, x_map_gene, Win_gene, bin_gene, Win_atac, bin_atac,
           Wl_tg, Wr_tg, b_tg, Wl_ag, Wr_ag, b_ag, Wl_pg, Wr_pg, b_pg,
           Wd1, bd1, Wd2, bd2, Wv1, bv1, Wv2, bv2):
    xg = _proj(x_gene, Win_gene, bin_gene)
    xa = _proj(x_atac, Win_atac, bin_atac)
    stg, dtg = _prep_edges(ei_tad_gene)
    sag, dag = _prep_edges(ei_atac_gene)
    spg, dpg = _prep_edges(ei_prot_gene)
    sum_tg, cnt_tg, sum_ag, cnt_ag, sum_pg, cnt_pg = _sc_segsum(
        stg, dtg, x_tad, sag, dag, xa, spg, dpg, x_protein
    )
    bs = (b_tg + b_ag + b_pg).reshape(1, D)
    p, v = _final(
        xg, sum_tg[:NG], cnt_tg[:NG], sum_ag[:NG], cnt_ag[:NG],
        sum_pg[:NG], cnt_pg[:NG],
        Wl_tg, Wl_ag, Wl_pg, Wr_tg, Wr_ag, Wr_pg, bs,
        Wd1, bd1.reshape(1, 64), Wd2, bd2.reshape(1, 1),
        Wv1, bv1.reshape(1, 64), Wv2, bv2.reshape(1, 1),
    )
    dropouts = jax.random.bernoulli(jax.random.key(1), p).astype(jnp.float32)
    return (p, dropouts, v, x_map_gene)
